# Initial kernel scaffold; baseline (speedup 1.0000x reference)
import functools

import jax
import jax.numpy as jnp
from jax import lax
from jax.experimental import pallas as pl
from jax.experimental.pallas import tpu as pltpu
from jax.experimental.pallas import tpu_sc as plsc

E = 160000
D = 256
H = 128
B = 64
NB = E // B
NC = 2
NS = 16
NW = NC * NS
KMAX = (NB + NW - 1) // NW


def _body(lig_hbm, prot_hbm, src_hbm, dst_hbm, out_hbm,
          idx_s, idx_d, ligr, protr, outb, sem1, sem2):
    w = lax.axis_index("s") * NC + lax.axis_index("c")
    lanes = lax.iota(jnp.int32, (16,))
    zeros16 = jnp.zeros((16,), jnp.int32)
    ones16 = jnp.ones((16,), jnp.int32)

    def do_block(k, carry):
        bid = w + NW * k

        @pl.when(bid < NB)
        def _():
            base = bid * B
            pltpu.sync_copy(src_hbm.at[pl.ds(base, B)], idx_s)
            pltpu.sync_copy(dst_hbm.at[pl.ds(base, B)], idx_d)
            cp1 = pltpu.async_copy(lig_hbm.at[idx_s], ligr, sem1)
            cp2 = pltpu.async_copy(prot_hbm.at[idx_d], protr, sem2)
            cp1.wait()
            cp2.wait()
            for g in range(B // 16):
                el = lanes + g * 16

                def d_abs(d, acc):
                    dd = jnp.full((16,), d, jnp.int32)
                    l = plsc.load_gather(ligr, [el, dd])
                    p = plsc.load_gather(protr, [el, dd])
                    return acc + jnp.abs(l - p)

                var = lax.fori_loop(0, H, d_abs,
                                    jnp.zeros((16,), jnp.float32), unroll=8)

                def d_sq(d, acc):
                    dd = jnp.full((16,), d, jnp.int32)
                    l = plsc.load_gather(ligr, [el, dd])
                    p = plsc.load_gather(protr, [el, dd])
                    df = l - p
                    return acc + df * df

                ss = lax.fori_loop(H, D, d_sq,
                                   jnp.zeros((16,), jnp.float32), unroll=8)

                i32 = plsc.bitcast(ss, jnp.int32)
                i32 = jnp.int32(0x5F3759DF) - lax.shift_right_arithmetic(i32, 1)
                y = plsc.bitcast(i32, jnp.float32)
                for _ in range(3):
                    y = y * (1.5 - 0.5 * ss * y * y)
                mu = jnp.where(ss > 0.0, ss * y, 0.0)

                plsc.store_scatter(outb, [el, zeros16], mu)
                plsc.store_scatter(outb, [el, ones16], var)
            pltpu.sync_copy(outb, out_hbm.at[pl.ds(base, B)])
        return carry

    lax.fori_loop(0, KMAX, do_block, 0)


@functools.partial(
    pl.kernel,
    out_type=jax.ShapeDtypeStruct((E, 2), jnp.float32),
    mesh=plsc.VectorSubcoreMesh(core_axis_name="c", subcore_axis_name="s"),
    scratch_types=[
        pltpu.VMEM((B,), jnp.int32),
        pltpu.VMEM((B,), jnp.int32),
        pltpu.VMEM((B, D), jnp.float32),
        pltpu.VMEM((B, D), jnp.float32),
        pltpu.VMEM((B, 2), jnp.float32),
        pltpu.SemaphoreType.DMA,
        pltpu.SemaphoreType.DMA,
    ],
)
def _sc_kernel(lig, prot, src, dst, out, idx_s, idx_d, ligr, protr, outb,
               sem1, sem2):
    _body(lig, prot, src, dst, out, idx_s, idx_d, ligr, protr, outb,
          sem1, sem2)


def kernel(ligand_h, protein_h, edge_index):
    ei = edge_index.astype(jnp.int32)
    return _sc_kernel(ligand_h, protein_h, ei[0], ei[1])


# speedup vs baseline: 2.0232x; 2.0232x over previous
import functools

import jax
import jax.numpy as jnp
from jax import lax
from jax.experimental import pallas as pl
from jax.experimental.pallas import tpu as pltpu
from jax.experimental.pallas import tpu_sc as plsc

E = 160000
D = 256
H = 128
B = 64
NB = E // B
NC = 2
NS = 16
NW = NC * NS
KMAX = (NB + NW - 1) // NW


def _body(lig_hbm, prot_hbm, src_hbm, dst_hbm, mu_hbm, var_hbm,
          idx_s, idx_d, ligr, protr, mub, varb, sem1, sem2):
    w = lax.axis_index("s") * NC + lax.axis_index("c")
    lanes = lax.iota(jnp.int32, 16)

    def do_block(k, carry):
        bid = w + NW * k

        @pl.when(bid < NB)
        def _():
            base = bid * B
            pltpu.sync_copy(src_hbm.at[pl.ds(base, B)], idx_s)
            pltpu.sync_copy(dst_hbm.at[pl.ds(base, B)], idx_d)
            cp1 = pltpu.async_copy(lig_hbm.at[idx_s], ligr, sem1)
            cp2 = pltpu.async_copy(prot_hbm.at[idx_d], protr, sem2)
            cp1.wait()
            cp2.wait()

            def group_body(g, carry2):
                var_acc = jnp.zeros((16,), jnp.float32)
                ss_acc = jnp.zeros((16,), jnp.float32)
                for m in range(16):
                    e = g * 16 + m
                    accv = jnp.zeros((16,), jnp.float32)
                    for j in range(H // 16):
                        l = ligr[e, pl.ds(j * 16, 16)]
                        p = protr[e, pl.ds(j * 16, 16)]
                        accv = accv + jnp.abs(l - p)
                    var = jnp.sum(accv)
                    accs = jnp.zeros((16,), jnp.float32)
                    for j in range(H // 16, D // 16):
                        l = ligr[e, pl.ds(j * 16, 16)]
                        p = protr[e, pl.ds(j * 16, 16)]
                        df = l - p
                        accs = accs + df * df
                    ss = jnp.sum(accs)
                    msk = lanes == m
                    var_acc = jnp.where(msk, var, var_acc)
                    ss_acc = jnp.where(msk, ss, ss_acc)

                i32 = plsc.bitcast(ss_acc, jnp.int32)
                i32 = jnp.int32(0x5F3759DF) - lax.shift_right_arithmetic(i32, 1)
                y = plsc.bitcast(i32, jnp.float32)
                for _ in range(3):
                    y = y * (1.5 - 0.5 * ss_acc * y * y)
                mu_vec = jnp.where(ss_acc > 0.0, ss_acc * y, 0.0)

                mub[pl.ds(g * 16, 16)] = mu_vec
                varb[pl.ds(g * 16, 16)] = var_acc
                return carry2

            lax.fori_loop(0, B // 16, group_body, 0)
            pltpu.sync_copy(mub, mu_hbm.at[pl.ds(base, B)])
            pltpu.sync_copy(varb, var_hbm.at[pl.ds(base, B)])
        return carry

    lax.fori_loop(0, KMAX, do_block, 0)


@functools.partial(
    pl.kernel,
    out_type=(
        jax.ShapeDtypeStruct((E,), jnp.float32),
        jax.ShapeDtypeStruct((E,), jnp.float32),
    ),
    mesh=plsc.VectorSubcoreMesh(core_axis_name="c", subcore_axis_name="s"),
    compiler_params=pltpu.CompilerParams(needs_layout_passes=False),
    scratch_types=[
        pltpu.VMEM((B,), jnp.int32),
        pltpu.VMEM((B,), jnp.int32),
        pltpu.VMEM((B, D), jnp.float32),
        pltpu.VMEM((B, D), jnp.float32),
        pltpu.VMEM((B,), jnp.float32),
        pltpu.VMEM((B,), jnp.float32),
        pltpu.SemaphoreType.DMA,
        pltpu.SemaphoreType.DMA,
    ],
)
def _sc_kernel(lig, prot, src, dst, mu_out, var_out,
               idx_s, idx_d, ligr, protr, mub, varb, sem1, sem2):
    _body(lig, prot, src, dst, mu_out, var_out,
          idx_s, idx_d, ligr, protr, mub, varb, sem1, sem2)


def kernel(ligand_h, protein_h, edge_index):
    ei = edge_index.astype(jnp.int32)
    mu, var = _sc_kernel(ligand_h, protein_h, ei[0], ei[1])
    return jnp.stack([mu, var], axis=0).T


# speedup vs baseline: 3.4321x; 1.6964x over previous
import functools

import jax
import jax.numpy as jnp
from jax import lax
from jax.experimental import pallas as pl
from jax.experimental.pallas import tpu as pltpu
from jax.experimental.pallas import tpu_sc as plsc

E = 160000
D = 256
H = 128
NC = 2
NS = 16
NW = NC * NS
EPW = E // NW
B = 96
NBW = EPW // B
TAIL = EPW - NBW * B
TPAD = 16
EBUF = EPW + (TPAD - TAIL)


def _body(lig_hbm, prot_hbm, src_hbm, dst_hbm, mu_hbm, var_hbm,
          idxs, idxd, ligA, protA, ligB, protB, mub, varb,
          sla, spa, slb, spb):
    w = lax.axis_index("s") * NC + lax.axis_index("c")
    ebase = w * EPW
    lanes = lax.iota(jnp.int32, 16)

    pltpu.sync_copy(src_hbm.at[pl.ds(ebase, EPW)], idxs.at[pl.ds(0, EPW)])
    pltpu.sync_copy(dst_hbm.at[pl.ds(ebase, EPW)], idxd.at[pl.ds(0, EPW)])
    vs = idxs[pl.ds(EBUF - 16, 16)]
    idxs[pl.ds(EBUF - 16, 16)] = jnp.where(lanes < TAIL, vs, 0)
    vd = idxd[pl.ds(EBUF - 16, 16)]
    idxd[pl.ds(EBUF - 16, 16)] = jnp.where(lanes < TAIL, vd, 0)

    def issue(off, n, ligbuf, protbuf, sem_l, sem_p):
        cl = pltpu.async_copy(lig_hbm.at[idxs.at[pl.ds(off, n)]],
                              ligbuf, sem_l)
        cp = pltpu.async_copy(prot_hbm.at[idxd.at[pl.ds(off, n)]],
                              protbuf, sem_p)
        return cl, cp

    def wait(off, n, ligbuf, protbuf, sem_l, sem_p):
        pltpu.make_async_copy(lig_hbm.at[idxs.at[pl.ds(off, n)]],
                              ligbuf, sem_l).wait()
        pltpu.make_async_copy(prot_hbm.at[idxd.at[pl.ds(off, n)]],
                              protbuf, sem_p).wait()

    def compute_block(obase, ligbuf, protbuf, ngroups):
        def group_body(g, carry):
            var_acc = jnp.zeros((16,), jnp.float32)
            ss_acc = jnp.zeros((16,), jnp.float32)
            for m in range(16):
                e = g * 16 + m
                accv = jnp.zeros((16,), jnp.float32)
                for j in range(H // 16):
                    l = ligbuf[e, pl.ds(j * 16, 16)]
                    p = protbuf[e, pl.ds(j * 16, 16)]
                    accv = accv + jnp.abs(l - p)
                var = jnp.sum(accv)
                accs = jnp.zeros((16,), jnp.float32)
                for j in range(H // 16, D // 16):
                    l = ligbuf[e, pl.ds(j * 16, 16)]
                    p = protbuf[e, pl.ds(j * 16, 16)]
                    df = l - p
                    accs = accs + df * df
                ss = jnp.sum(accs)
                msk = lanes == m
                var_acc = jnp.where(msk, var, var_acc)
                ss_acc = jnp.where(msk, ss, ss_acc)

            i32 = plsc.bitcast(ss_acc, jnp.int32)
            i32 = jnp.int32(0x5F3759DF) - lax.shift_right_arithmetic(i32, 1)
            y = plsc.bitcast(i32, jnp.float32)
            for _ in range(3):
                y = y * (1.5 - 0.5 * ss_acc * y * y)
            mu_vec = jnp.where(ss_acc > 0.0, ss_acc * y, 0.0)

            mub[pl.ds(obase + g * 16, 16)] = mu_vec
            varb[pl.ds(obase + g * 16, 16)] = var_acc
            return carry

        lax.fori_loop(0, ngroups, group_body, 0)

    issue(0, B, ligA, protA, sla, spa)

    def pair_body(p, carry):
        k = 2 * p
        issue((k + 1) * B, B, ligB, protB, slb, spb)
        wait(k * B, B, ligA, protA, sla, spa)
        compute_block(k * B, ligA, protA, B // 16)

        @pl.when(k + 2 < NBW)
        def _():
            issue((k + 2) * B, B, ligA, protA, sla, spa)

        wait((k + 1) * B, B, ligB, protB, slb, spb)
        compute_block((k + 1) * B, ligB, protB, B // 16)
        return carry

    lax.fori_loop(0, NBW // 2, pair_body, 0)

    tl = ligA.at[pl.ds(0, TPAD), :]
    tp = protA.at[pl.ds(0, TPAD), :]
    issue(NBW * B, TPAD, tl, tp, sla, spa)
    wait(NBW * B, TPAD, tl, tp, sla, spa)
    compute_block(NBW * B, ligA, protA, 1)

    pltpu.sync_copy(mub.at[pl.ds(0, EPW)], mu_hbm.at[pl.ds(ebase, EPW)])
    pltpu.sync_copy(varb.at[pl.ds(0, EPW)], var_hbm.at[pl.ds(ebase, EPW)])


@functools.partial(
    pl.kernel,
    out_type=(
        jax.ShapeDtypeStruct((E,), jnp.float32),
        jax.ShapeDtypeStruct((E,), jnp.float32),
    ),
    mesh=plsc.VectorSubcoreMesh(core_axis_name="c", subcore_axis_name="s"),
    compiler_params=pltpu.CompilerParams(needs_layout_passes=False),
    scratch_types=[
        pltpu.VMEM((EBUF,), jnp.int32),
        pltpu.VMEM((EBUF,), jnp.int32),
        pltpu.VMEM((B, D), jnp.float32),
        pltpu.VMEM((B, D), jnp.float32),
        pltpu.VMEM((B, D), jnp.float32),
        pltpu.VMEM((B, D), jnp.float32),
        pltpu.VMEM((EBUF,), jnp.float32),
        pltpu.VMEM((EBUF,), jnp.float32),
        pltpu.SemaphoreType.DMA,
        pltpu.SemaphoreType.DMA,
        pltpu.SemaphoreType.DMA,
        pltpu.SemaphoreType.DMA,
    ],
)
def _sc_kernel(lig, prot, src, dst, mu_out, var_out,
               idxs, idxd, ligA, protA, ligB, protB, mub, varb,
               sla, spa, slb, spb):
    _body(lig, prot, src, dst, mu_out, var_out,
          idxs, idxd, ligA, protA, ligB, protB, mub, varb,
          sla, spa, slb, spb)


def kernel(ligand_h, protein_h, edge_index):
    ei = edge_index.astype(jnp.int32)
    mu, var = _sc_kernel(ligand_h, protein_h, ei[0], ei[1])
    return jnp.stack([mu, var], axis=0).T
